# baseline (speedup 1.0000x reference)
import jax
import jax.numpy as jnp
from jax.experimental import pallas as pl
from jax.experimental.pallas import tpu as pltpu

N_TOK = 8192
N_FEAT = 128
N_SUPER = 32
N_BATCH = 8
BLK = 8192
NBLK = N_TOK // BLK
_HI = jax.lax.Precision.HIGHEST
_DEF = jax.lax.Precision.DEFAULT


def _fused_kernel(x_ref, b_ref, aug_ref, w1_ref, b1_ref, w2_ref, b2_ref,
                  sc_ref, g_ref, am_ref,
                  s_out, out_ref, mu_ref, aux_ref,
                  denomT_ref, ent_ref):
    i = pl.program_id(0)

    @pl.when(i == 0)
    def _init():
        out_ref[...] = jnp.zeros_like(out_ref)
        mu_ref[...] = jnp.zeros_like(mu_ref)
        denomT_ref[...] = jnp.zeros_like(denomT_ref)
        aux_ref[...] = jnp.zeros_like(aux_ref)
        ent_ref[0, 0] = 0.0

    x = x_ref[...]
    hT = jnp.maximum(
        jax.lax.dot_general(w1_ref[...], x, (((0,), (1,)), ((), ())),
                            precision=_DEF) + b1_ref[...], 0.0)
    logitsT = (jax.lax.dot_general(w2_ref[...], hT, (((0,), (0,)), ((), ())),
                                   precision=_DEF)
               + b2_ref[...]) * sc_ref[0, 0]
    logitsT = jnp.where(am_ref[...] == 0.0, -1e9, logitsT)
    gT = jnp.transpose(g_ref[...])
    zT = logitsT + gT
    m = jnp.max(zT, axis=0, keepdims=True)
    eT = jnp.exp(zT - m)
    sT = eT / jnp.sum(eT, axis=0, keepdims=True)
    s_out[...] = jnp.transpose(sT)

    rho = jnp.sum(sT * jnp.log(sT + 1e-9), axis=0, keepdims=True)
    ent_ref[0, 0] += jnp.sum(rho)

    brow = b_ref[...]
    aug = aug_ref[...]
    blo = b_ref[0, 0]
    bhi = b_ref[0, BLK - 1]

    def _accum_rows(sbT):
        numb = jax.lax.dot_general(sbT, x, (((1,), (0,)), ((), ())),
                                   precision=_DEF)
        auxr = jax.lax.dot_general(sbT, aug, (((1,), (1,)), ((), ())),
                                   precision=_DEF)
        return numb, auxr

    batch_iota = jax.lax.broadcasted_iota(jnp.int32, (1, N_BATCH), 1)

    @pl.when(blo == bhi)
    def _single():
        numb, auxr = _accum_rows(sT)
        bs = pl.ds(blo, 1)
        out_ref[bs, :, :] += numb.reshape(1, N_SUPER, N_FEAT)
        oneh = (batch_iota == blo).astype(jnp.float32)
        denomT_ref[...] += auxr[:, 0:1] * oneh
        mu_ref[bs, :, :] += auxr[:, 1:3].reshape(1, N_SUPER, 2)

    @pl.when(blo != bhi)
    def _multi():
        for b in range(N_BATCH):
            @pl.when((b >= blo) & (b <= bhi))
            def _pool():
                mask = (brow == b).astype(jnp.float32)
                numb, auxr = _accum_rows(sT * mask)
                out_ref[b, :, :] += numb
                oneh = (batch_iota == b).astype(jnp.float32)
                denomT_ref[...] += auxr[:, 0:1] * oneh
                mu_ref[b, :, :] += auxr[:, 1:3]

    @pl.when(i == NBLK - 1)
    def _epilogue():
        K = N_SUPER
        dT = jnp.transpose(denomT_ref[...])
        dsafe = dT + 1e-9
        out_ref[...] = out_ref[...] / dsafe[:, :, None]
        mu_ref[...] = mu_ref[...] / dsafe[:, :, None]

        avg2 = jnp.sum(dT, axis=0, keepdims=True) / N_TOK
        entropy = -ent_ref[0, 0] / N_TOK
        u = 1.0 / K
        diversity = jnp.sum(u * (jnp.log(u) - jnp.log(avg2 + 1e-9)))
        am2 = jnp.transpose(am_ref[...])
        pruning = jnp.mean(jnp.abs(avg2 * (1.0 - am2)))
        sparsity = jnp.sum(am2) / K
        collapse = jnp.maximum(jnp.max(avg2) - u, 0.0)
        balance = jnp.sum((avg2 - u) ** 2) * K

        eyec = 1.0 - jnp.eye(K, dtype=jnp.float32)
        ones12 = jnp.ones((1, 2), dtype=jnp.float32)
        sep = 0.0
        for b in range(N_BATCH):
            mub = mu_ref[b]
            sq = mub * mub
            n2col = jnp.sum(sq, axis=1, keepdims=True)
            n2row = jax.lax.dot_general(ones12, sq,
                                        (((1,), (1,)), ((), ())),
                                        precision=_HI)
            G = jax.lax.dot_general(mub, mub,
                                    (((1,), (1,)), ((), ())),
                                    precision=_HI)
            d = n2col + n2row - 2.0 * G
            sep += jnp.sum(jnp.exp(-d) * eyec)
        separation = sep / (N_BATCH * K * K)

        aux_ref[0:1, 0:1] = jnp.reshape(entropy, (1, 1))
        aux_ref[0:1, 1:2] = jnp.reshape(diversity, (1, 1))
        aux_ref[0:1, 2:3] = jnp.reshape(pruning, (1, 1))
        aux_ref[0:1, 3:4] = jnp.reshape(sparsity, (1, 1))
        aux_ref[0:1, 4:5] = jnp.reshape(collapse, (1, 1))
        aux_ref[0:1, 5:6] = jnp.reshape(balance, (1, 1))
        aux_ref[0:1, 6:7] = jnp.reshape(separation, (1, 1))


@jax.jit
def _run(x, batch1n, aug, W1, b1c, W2, b2c, scaling2d, gumbel, amc):
    grid = (NBLK,)
    out_shapes = [
        jax.ShapeDtypeStruct((N_TOK, N_SUPER), jnp.float32),
        jax.ShapeDtypeStruct((N_BATCH, N_SUPER, N_FEAT), jnp.float32),
        jax.ShapeDtypeStruct((N_BATCH, N_SUPER, 2), jnp.float32),
        jax.ShapeDtypeStruct((8, 128), jnp.float32),
    ]
    in_specs = [
        pl.BlockSpec((BLK, N_FEAT), lambda i: (i, 0)),
        pl.BlockSpec((1, BLK), lambda i: (0, i)),
        pl.BlockSpec((3, BLK), lambda i: (0, i)),
        pl.BlockSpec((N_FEAT, N_FEAT), lambda i: (0, 0)),
        pl.BlockSpec((N_FEAT, 1), lambda i: (0, 0)),
        pl.BlockSpec((N_FEAT, N_SUPER), lambda i: (0, 0)),
        pl.BlockSpec((N_SUPER, 1), lambda i: (0, 0)),
        pl.BlockSpec(memory_space=pltpu.SMEM),
        pl.BlockSpec((BLK, N_SUPER), lambda i: (i, 0)),
        pl.BlockSpec((N_SUPER, 1), lambda i: (0, 0)),
    ]
    out_specs = [
        pl.BlockSpec((BLK, N_SUPER), lambda i: (i, 0)),
        pl.BlockSpec((N_BATCH, N_SUPER, N_FEAT), lambda i: (0, 0, 0)),
        pl.BlockSpec((N_BATCH, N_SUPER, 2), lambda i: (0, 0, 0)),
        pl.BlockSpec((8, 128), lambda i: (0, 0)),
    ]
    scratch = [
        pltpu.VMEM((N_SUPER, N_BATCH), jnp.float32),
        pltpu.SMEM((1, 1), jnp.float32),
    ]
    return pl.pallas_call(
        _fused_kernel,
        grid=grid,
        in_specs=in_specs,
        out_specs=out_specs,
        out_shape=out_shapes,
        scratch_shapes=scratch,
        compiler_params=pltpu.CompilerParams(
            dimension_semantics=("arbitrary",)),
    )(x, batch1n, aug, W1, b1c, W2, b2c, scaling2d, gumbel, amc)


def kernel(x, batch, pos, W1, b1, W2, b2, scaling, gumbel, active_mask):
    batch1n = batch.reshape(1, N_TOK)
    aug = jnp.concatenate(
        [jnp.ones((1, N_TOK), jnp.float32), pos.T], axis=0)
    s, out, mu, aux = _run(
        x, batch1n, aug, W1, b1.reshape(N_FEAT, 1), W2,
        b2.reshape(N_SUPER, 1), scaling.reshape(1, 1), gumbel,
        active_mask.reshape(N_SUPER, 1))
    entropy = aux[0, 0]
    diversity = aux[0, 1]
    pruning = aux[0, 2]
    sparsity = aux[0, 3]
    collapse = aux[0, 4]
    balance = aux[0, 5]
    separation = aux[0, 6]
    zero = jnp.zeros((), jnp.float32)
    return (out, s, mu, entropy, diversity, zero, pruning, sparsity,
            zero, collapse, balance, separation)


# speedup vs baseline: 1.0990x; 1.0990x over previous
import jax
import jax.numpy as jnp
from jax.experimental import pallas as pl
from jax.experimental.pallas import tpu as pltpu

N_TOK = 8192
N_FEAT = 128
N_SUPER = 32
N_BATCH = 8
BLK = 2048
NBLK = N_TOK // BLK
_HI = jax.lax.Precision.HIGHEST
_DEF = jax.lax.Precision.DEFAULT


def _fused_kernel(x_ref, b_ref, aug_ref, w1_ref, b1_ref, w2_ref, b2_ref,
                  sc_ref, g_ref, am_ref,
                  s_out, out_ref, mu_ref, aux_ref,
                  denomT_ref, ent_ref):
    i = pl.program_id(0)

    @pl.when(i == 0)
    def _init():
        out_ref[...] = jnp.zeros_like(out_ref)
        mu_ref[...] = jnp.zeros_like(mu_ref)
        denomT_ref[...] = jnp.zeros_like(denomT_ref)
        aux_ref[...] = jnp.zeros_like(aux_ref)
        ent_ref[0, 0] = 0.0

    x = x_ref[...]
    hT = jnp.maximum(
        jax.lax.dot_general(w1_ref[...], x, (((0,), (1,)), ((), ())),
                            precision=_DEF) + b1_ref[...], 0.0)
    logitsT = (jax.lax.dot_general(w2_ref[...], hT, (((0,), (0,)), ((), ())),
                                   precision=_DEF)
               + b2_ref[...]) * sc_ref[0, 0]
    logitsT = jnp.where(am_ref[...] == 0.0, -1e9, logitsT)
    zT = logitsT + g_ref[...]
    m = jnp.max(zT, axis=0, keepdims=True)
    eT = jnp.exp(zT - m)
    sT = eT / jnp.sum(eT, axis=0, keepdims=True)
    s_out[...] = jnp.transpose(sT)

    rho = jnp.sum(sT * jnp.log(sT + 1e-9), axis=0, keepdims=True)
    ent_ref[0, 0] += jnp.sum(rho)

    brow = b_ref[...]
    aug = aug_ref[...]
    blo = b_ref[0, 0]
    bhi = b_ref[0, BLK - 1]

    def _accum_rows(sbT):
        numb = jax.lax.dot_general(sbT, x, (((1,), (0,)), ((), ())),
                                   precision=_DEF)
        auxr = jax.lax.dot_general(sbT, aug, (((1,), (1,)), ((), ())),
                                   precision=_DEF)
        return numb, auxr

    batch_iota = jax.lax.broadcasted_iota(jnp.int32, (1, N_BATCH), 1)

    @pl.when(blo == bhi)
    def _single():
        numb, auxr = _accum_rows(sT)
        bs = pl.ds(blo, 1)
        out_ref[bs, :, :] += numb.reshape(1, N_SUPER, N_FEAT)
        oneh = (batch_iota == blo).astype(jnp.float32)
        denomT_ref[...] += auxr[:, 0:1] * oneh
        mu_ref[bs, :, :] += auxr[:, 1:3].reshape(1, N_SUPER, 2)

    @pl.when(blo != bhi)
    def _multi():
        for b in range(N_BATCH):
            @pl.when((b >= blo) & (b <= bhi))
            def _pool():
                mask = (brow == b).astype(jnp.float32)
                numb, auxr = _accum_rows(sT * mask)
                out_ref[b, :, :] += numb
                oneh = (batch_iota == b).astype(jnp.float32)
                denomT_ref[...] += auxr[:, 0:1] * oneh
                mu_ref[b, :, :] += auxr[:, 1:3]

    @pl.when(i == NBLK - 1)
    def _epilogue():
        K = N_SUPER
        dT = jnp.transpose(denomT_ref[...])
        dsafe = dT + 1e-9
        out_ref[...] = out_ref[...] / dsafe[:, :, None]
        mu_ref[...] = mu_ref[...] / dsafe[:, :, None]

        avg2 = jnp.sum(dT, axis=0, keepdims=True) / N_TOK
        entropy = -ent_ref[0, 0] / N_TOK
        u = 1.0 / K
        diversity = jnp.sum(u * (jnp.log(u) - jnp.log(avg2 + 1e-9)))
        am2 = jnp.transpose(am_ref[...])
        pruning = jnp.mean(jnp.abs(avg2 * (1.0 - am2)))
        sparsity = jnp.sum(am2) / K
        collapse = jnp.maximum(jnp.max(avg2) - u, 0.0)
        balance = jnp.sum((avg2 - u) ** 2) * K

        eyec = 1.0 - jnp.eye(K, dtype=jnp.float32)
        ones12 = jnp.ones((1, 2), dtype=jnp.float32)
        sep = 0.0
        for b in range(N_BATCH):
            mub = mu_ref[b]
            sq = mub * mub
            n2col = jnp.sum(sq, axis=1, keepdims=True)
            n2row = jax.lax.dot_general(ones12, sq,
                                        (((1,), (1,)), ((), ())),
                                        precision=_HI)
            G = jax.lax.dot_general(mub, mub,
                                    (((1,), (1,)), ((), ())),
                                    precision=_HI)
            d = n2col + n2row - 2.0 * G
            sep += jnp.sum(jnp.exp(-d) * eyec)
        separation = sep / (N_BATCH * K * K)

        aux_ref[0:1, 0:1] = jnp.reshape(entropy, (1, 1))
        aux_ref[0:1, 1:2] = jnp.reshape(diversity, (1, 1))
        aux_ref[0:1, 2:3] = jnp.reshape(pruning, (1, 1))
        aux_ref[0:1, 3:4] = jnp.reshape(sparsity, (1, 1))
        aux_ref[0:1, 4:5] = jnp.reshape(collapse, (1, 1))
        aux_ref[0:1, 5:6] = jnp.reshape(balance, (1, 1))
        aux_ref[0:1, 6:7] = jnp.reshape(separation, (1, 1))


@jax.jit
def _run(x, batch1n, aug, W1, b1c, W2, b2c, scaling2d, gumbel, amc):
    grid = (NBLK,)
    out_shapes = [
        jax.ShapeDtypeStruct((N_TOK, N_SUPER), jnp.float32),
        jax.ShapeDtypeStruct((N_BATCH, N_SUPER, N_FEAT), jnp.float32),
        jax.ShapeDtypeStruct((N_BATCH, N_SUPER, 2), jnp.float32),
        jax.ShapeDtypeStruct((8, 128), jnp.float32),
    ]
    in_specs = [
        pl.BlockSpec((BLK, N_FEAT), lambda i: (i, 0)),
        pl.BlockSpec((1, BLK), lambda i: (0, i)),
        pl.BlockSpec((3, BLK), lambda i: (0, i)),
        pl.BlockSpec((N_FEAT, N_FEAT), lambda i: (0, 0)),
        pl.BlockSpec((N_FEAT, 1), lambda i: (0, 0)),
        pl.BlockSpec((N_FEAT, N_SUPER), lambda i: (0, 0)),
        pl.BlockSpec((N_SUPER, 1), lambda i: (0, 0)),
        pl.BlockSpec(memory_space=pltpu.SMEM),
        pl.BlockSpec((N_SUPER, BLK), lambda i: (0, i)),
        pl.BlockSpec((N_SUPER, 1), lambda i: (0, 0)),
    ]
    out_specs = [
        pl.BlockSpec((BLK, N_SUPER), lambda i: (i, 0)),
        pl.BlockSpec((N_BATCH, N_SUPER, N_FEAT), lambda i: (0, 0, 0)),
        pl.BlockSpec((N_BATCH, N_SUPER, 2), lambda i: (0, 0, 0)),
        pl.BlockSpec((8, 128), lambda i: (0, 0)),
    ]
    scratch = [
        pltpu.VMEM((N_SUPER, N_BATCH), jnp.float32),
        pltpu.SMEM((1, 1), jnp.float32),
    ]
    return pl.pallas_call(
        _fused_kernel,
        grid=grid,
        in_specs=in_specs,
        out_specs=out_specs,
        out_shape=out_shapes,
        scratch_shapes=scratch,
        compiler_params=pltpu.CompilerParams(
            dimension_semantics=("arbitrary",)),
    )(x, batch1n, aug, W1, b1c, W2, b2c, scaling2d, gumbel, amc)


def kernel(x, batch, pos, W1, b1, W2, b2, scaling, gumbel, active_mask):
    batch1n = batch.reshape(1, N_TOK)
    aug = jnp.concatenate(
        [jnp.ones((1, N_TOK), jnp.float32), pos.T], axis=0)
    s, out, mu, aux = _run(
        x, batch1n, aug, W1, b1.reshape(N_FEAT, 1), W2,
        b2.reshape(N_SUPER, 1), scaling.reshape(1, 1), gumbel.T,
        active_mask.reshape(N_SUPER, 1))
    entropy = aux[0, 0]
    diversity = aux[0, 1]
    pruning = aux[0, 2]
    sparsity = aux[0, 3]
    collapse = aux[0, 4]
    balance = aux[0, 5]
    separation = aux[0, 6]
    zero = jnp.zeros((), jnp.float32)
    return (out, s, mu, entropy, diversity, zero, pruning, sparsity,
            zero, collapse, balance, separation)


# speedup vs baseline: 1.0990x; 1.0000x over previous
import jax
import jax.numpy as jnp
from jax.experimental import pallas as pl
from jax.experimental.pallas import tpu as pltpu

N_TOK = 8192
N_FEAT = 128
N_SUPER = 32
N_BATCH = 8
BLK = 2048
NBLK = N_TOK // BLK
_HI = jax.lax.Precision.HIGHEST
_DEF = jax.lax.Precision.DEFAULT


def _fused_kernel(x_ref, b_ref, aug_ref, w1_ref, b1_ref, w2_ref, b2_ref,
                  sc_ref, g_ref, am_ref,
                  s_out, out_ref, mu_ref, aux_ref,
                  denomT_ref, ent_ref):
    i = pl.program_id(0)

    @pl.when(i == 0)
    def _init():
        out_ref[...] = jnp.zeros_like(out_ref)
        mu_ref[...] = jnp.zeros_like(mu_ref)
        denomT_ref[...] = jnp.zeros_like(denomT_ref)
        aux_ref[...] = jnp.zeros_like(aux_ref)
        ent_ref[0, 0] = 0.0

    x = x_ref[...]
    hT = jnp.maximum(
        jax.lax.dot_general(w1_ref[...], x, (((0,), (1,)), ((), ())),
                            precision=_DEF) + b1_ref[...], 0.0)
    logitsT = (jax.lax.dot_general(w2_ref[...], hT, (((0,), (0,)), ((), ())),
                                   precision=_DEF)
               + b2_ref[...]) * sc_ref[0, 0]
    logitsT = jnp.where(am_ref[...] == 0.0, -1e9, logitsT)
    zT = logitsT + g_ref[...]
    m = jnp.max(zT, axis=0, keepdims=True)
    eT = jnp.exp(zT - m)
    sumE = jnp.sum(eT, axis=0, keepdims=True)
    sT = eT / sumE
    s_out[...] = jnp.transpose(sT)

    rho = (jnp.sum(sT * zT, axis=0, keepdims=True) - m - jnp.log(sumE))
    ent_ref[0, 0] += jnp.sum(rho)

    brow = b_ref[...]
    aug = aug_ref[...]
    blo = b_ref[0, 0]
    bhi = b_ref[0, BLK - 1]

    def _accum_rows(sbT):
        numb = jax.lax.dot_general(sbT, x, (((1,), (0,)), ((), ())),
                                   precision=_DEF)
        auxr = jax.lax.dot_general(sbT, aug, (((1,), (1,)), ((), ())),
                                   precision=_DEF)
        return numb, auxr

    batch_iota = jax.lax.broadcasted_iota(jnp.int32, (1, N_BATCH), 1)

    @pl.when(blo == bhi)
    def _single():
        numb, auxr = _accum_rows(sT)
        bs = pl.ds(blo, 1)
        out_ref[bs, :, :] += numb.reshape(1, N_SUPER, N_FEAT)
        oneh = (batch_iota == blo).astype(jnp.float32)
        denomT_ref[...] += auxr[:, 0:1] * oneh
        mu_ref[bs, :, :] += auxr[:, 1:3].reshape(1, N_SUPER, 2)

    @pl.when(blo != bhi)
    def _multi():
        for b in range(N_BATCH):
            @pl.when((b >= blo) & (b <= bhi))
            def _pool():
                mask = (brow == b).astype(jnp.float32)
                numb, auxr = _accum_rows(sT * mask)
                out_ref[b, :, :] += numb
                oneh = (batch_iota == b).astype(jnp.float32)
                denomT_ref[...] += auxr[:, 0:1] * oneh
                mu_ref[b, :, :] += auxr[:, 1:3]

    @pl.when(i == NBLK - 1)
    def _epilogue():
        K = N_SUPER
        dT = jnp.transpose(denomT_ref[...])
        dsafe = dT + 1e-9
        out_ref[...] = out_ref[...] / dsafe[:, :, None]
        mu_ref[...] = mu_ref[...] / dsafe[:, :, None]

        avg2 = jnp.sum(dT, axis=0, keepdims=True) / N_TOK
        entropy = -ent_ref[0, 0] / N_TOK
        u = 1.0 / K
        diversity = jnp.sum(u * (jnp.log(u) - jnp.log(avg2 + 1e-9)))
        am2 = jnp.transpose(am_ref[...])
        pruning = jnp.mean(jnp.abs(avg2 * (1.0 - am2)))
        sparsity = jnp.sum(am2) / K
        collapse = jnp.maximum(jnp.max(avg2) - u, 0.0)
        balance = jnp.sum((avg2 - u) ** 2) * K

        eyec = 1.0 - jnp.eye(K, dtype=jnp.float32)
        ones12 = jnp.ones((1, 2), dtype=jnp.float32)
        sep = 0.0
        for b in range(N_BATCH):
            mub = mu_ref[b]
            sq = mub * mub
            n2col = jnp.sum(sq, axis=1, keepdims=True)
            n2row = jax.lax.dot_general(ones12, sq,
                                        (((1,), (1,)), ((), ())),
                                        precision=_HI)
            G = jax.lax.dot_general(mub, mub,
                                    (((1,), (1,)), ((), ())),
                                    precision=_HI)
            d = n2col + n2row - 2.0 * G
            sep += jnp.sum(jnp.exp(-d) * eyec)
        separation = sep / (N_BATCH * K * K)

        aux_ref[0:1, 0:1] = jnp.reshape(entropy, (1, 1))
        aux_ref[0:1, 1:2] = jnp.reshape(diversity, (1, 1))
        aux_ref[0:1, 2:3] = jnp.reshape(pruning, (1, 1))
        aux_ref[0:1, 3:4] = jnp.reshape(sparsity, (1, 1))
        aux_ref[0:1, 4:5] = jnp.reshape(collapse, (1, 1))
        aux_ref[0:1, 5:6] = jnp.reshape(balance, (1, 1))
        aux_ref[0:1, 6:7] = jnp.reshape(separation, (1, 1))


@jax.jit
def _run(x, batch1n, aug, W1, b1c, W2, b2c, scaling2d, gumbel, amc):
    grid = (NBLK,)
    out_shapes = [
        jax.ShapeDtypeStruct((N_TOK, N_SUPER), jnp.float32),
        jax.ShapeDtypeStruct((N_BATCH, N_SUPER, N_FEAT), jnp.float32),
        jax.ShapeDtypeStruct((N_BATCH, N_SUPER, 2), jnp.float32),
        jax.ShapeDtypeStruct((8, 128), jnp.float32),
    ]
    in_specs = [
        pl.BlockSpec((BLK, N_FEAT), lambda i: (i, 0)),
        pl.BlockSpec((1, BLK), lambda i: (0, i)),
        pl.BlockSpec((3, BLK), lambda i: (0, i)),
        pl.BlockSpec((N_FEAT, N_FEAT), lambda i: (0, 0)),
        pl.BlockSpec((N_FEAT, 1), lambda i: (0, 0)),
        pl.BlockSpec((N_FEAT, N_SUPER), lambda i: (0, 0)),
        pl.BlockSpec((N_SUPER, 1), lambda i: (0, 0)),
        pl.BlockSpec(memory_space=pltpu.SMEM),
        pl.BlockSpec((N_SUPER, BLK), lambda i: (0, i)),
        pl.BlockSpec((N_SUPER, 1), lambda i: (0, 0)),
    ]
    out_specs = [
        pl.BlockSpec((BLK, N_SUPER), lambda i: (i, 0)),
        pl.BlockSpec((N_BATCH, N_SUPER, N_FEAT), lambda i: (0, 0, 0)),
        pl.BlockSpec((N_BATCH, N_SUPER, 2), lambda i: (0, 0, 0)),
        pl.BlockSpec((8, 128), lambda i: (0, 0)),
    ]
    scratch = [
        pltpu.VMEM((N_SUPER, N_BATCH), jnp.float32),
        pltpu.SMEM((1, 1), jnp.float32),
    ]
    return pl.pallas_call(
        _fused_kernel,
        grid=grid,
        in_specs=in_specs,
        out_specs=out_specs,
        out_shape=out_shapes,
        scratch_shapes=scratch,
        compiler_params=pltpu.CompilerParams(
            dimension_semantics=("arbitrary",)),
    )(x, batch1n, aug, W1, b1c, W2, b2c, scaling2d, gumbel, amc)


def kernel(x, batch, pos, W1, b1, W2, b2, scaling, gumbel, active_mask):
    batch1n = batch.reshape(1, N_TOK)
    aug = jnp.concatenate(
        [jnp.ones((1, N_TOK), jnp.float32), pos.T], axis=0)
    s, out, mu, aux = _run(
        x, batch1n, aug, W1, b1.reshape(N_FEAT, 1), W2,
        b2.reshape(N_SUPER, 1), scaling.reshape(1, 1), gumbel.T,
        active_mask.reshape(N_SUPER, 1))
    entropy = aux[0, 0]
    diversity = aux[0, 1]
    pruning = aux[0, 2]
    sparsity = aux[0, 3]
    collapse = aux[0, 4]
    balance = aux[0, 5]
    separation = aux[0, 6]
    zero = jnp.zeros((), jnp.float32)
    return (out, s, mu, entropy, diversity, zero, pruning, sparsity,
            zero, collapse, balance, separation)
